# baseline (speedup 1.0000x reference)
import jax
import jax.numpy as jnp
from jax import lax
from jax.experimental import pallas as pl
from jax.experimental.pallas import tpu as pltpu
from jax.experimental.pallas import tpu_sc as plsc
from jax._src.pallas import mpmd as _mpmd

K_MAX = 100000
P_DIM = 64
MODEL_DIM = 64
N = 16384
RATE = 0.1

NC = 2
NS = 16
CHUNK = 128
PAD = 80
NCH = (N // CHUNK) // NS

_MESH = dict(core_axis_name="c", subcore_axis_name="s", num_cores=NC,
             num_subcores=NS)
_SC_PARAMS = pltpu.CompilerParams(use_tc_tiling_on_sc=False)


def _k0_body(src, dst):
    dst[...] = src[...]


def _k0(prototypes):
    blk = 2000
    return pl.pallas_call(
        _k0_body,
        grid=(K_MAX // blk,),
        in_specs=[pl.BlockSpec((blk, P_DIM), lambda i: (i, 0))],
        out_specs=pl.BlockSpec((blk, P_DIM), lambda i: (i, 0)),
        out_shape=jax.ShapeDtypeStruct((K_MAX, P_DIM), jnp.float32),
    )(prototypes)


def _k1_body(protos, ids2d, iota2d, pgath, slot, idx_v, iot_v, row_v,
             gsem, ssem, wsem):
    wid = lax.axis_index("c") * NS + lax.axis_index("s")
    nrow = (N // CHUNK) // (NC * NS)
    pltpu.sync_copy(ids2d.at[pl.ds(wid * nrow, nrow)], idx_v)
    pltpu.sync_copy(iota2d.at[pl.ds(wid * nrow, nrow)], iot_v)
    gets = [pltpu.async_copy(protos.at[idx_v.at[c]], row_v.at[c], gsem)
            for c in range(nrow)]
    puts = [pltpu.async_copy(iot_v.at[c], slot.at[idx_v.at[c]], ssem)
            for c in range(nrow)]
    wrs = []
    for c in range(nrow):
        gets[c].wait()
        base = (wid * nrow + c) * CHUNK
        wrs.append(pltpu.async_copy(row_v.at[c], pgath.at[pl.ds(base, CHUNK)],
                                    wsem))
    for d in puts + wrs:
        d.wait()


def _k1(prototypes, ids2d, iota2d):
    nrow = (N // CHUNK) // (NC * NS)
    return pl.kernel(
        _k1_body,
        out_type=(
            jax.ShapeDtypeStruct((N, P_DIM), jnp.float32),
            jax.ShapeDtypeStruct((K_MAX,), jnp.int32),
        ),
        mesh=plsc.VectorSubcoreMesh(**_MESH),
        compiler_params=_SC_PARAMS,
        scratch_types=[
            pltpu.VMEM((nrow, CHUNK), jnp.int32),
            pltpu.VMEM((nrow, CHUNK), jnp.int32),
            pltpu.VMEM((nrow, CHUNK, P_DIM), jnp.float32),
            pltpu.SemaphoreType.DMA,
            pltpu.SemaphoreType.DMA,
            pltpu.SemaphoreType.DMA,
        ],
    )(prototypes, ids2d, iota2d)


_DEPTH = 3


def _k2_body(ids2d, slot, values, zb, ones16, sums, acc, idx_v, rep_v, buf_v,
             s1, s2, s3):
    cid = lax.axis_index("c")
    w = lax.axis_index("s")
    nrow = (N // CHUNK) // NS

    @pl.when(cid == 0)
    def _zero():
        pltpu.sync_copy(zb, buf_v.at[0])
        zs = [pltpu.async_copy(
            buf_v.at[0], acc.at[pl.ds((w * nrow + j) * CHUNK, CHUNK)], s1)
            for j in range(nrow)]
        for d in zs:
            d.wait()

    plsc.subcore_barrier()

    @pl.when(cid == 0)
    def _accum():
        for d in range(_DEPTH):
            pltpu.sync_copy(ones16, buf_v.at[d, :, pl.ds(P_DIM, PAD - P_DIM)])
        pltpu.sync_copy(ids2d.at[pl.ds(w * nrow, nrow)], idx_v)
        reps = [pltpu.async_copy(slot.at[idx_v.at[c]], rep_v.at[c], s1)
                for c in range(nrow)]
        vals = {c: pltpu.async_copy(
            values.at[pl.ds((w * nrow + c) * CHUNK, CHUNK)],
            buf_v.at[c % _DEPTH, :, pl.ds(0, P_DIM)], s2)
            for c in range(_DEPTH)}
        adds = {}
        for c in range(nrow):
            reps[c].wait()
            vals[c].wait()
            adds[c] = pltpu.async_copy(buf_v.at[c % _DEPTH],
                                       acc.at[rep_v.at[c]], s3, add=True)
            nc = c + _DEPTH
            if nc < nrow:
                adds[c].wait()
                vals[nc] = pltpu.async_copy(
                    values.at[pl.ds((w * nrow + nc) * CHUNK, CHUNK)],
                    buf_v.at[nc % _DEPTH, :, pl.ds(0, P_DIM)], s2)
        for c in range(max(0, nrow - _DEPTH), nrow):
            adds[c].wait()

    plsc.subcore_barrier()

    @pl.when(cid == 0)
    def _readback():
        gets = {c: pltpu.async_copy(acc.at[rep_v.at[c]],
                                    buf_v.at[c % _DEPTH], s1)
                for c in range(_DEPTH)}
        wrs = {}
        for c in range(nrow):
            gets[c].wait()
            wrs[c] = pltpu.async_copy(
                buf_v.at[c % _DEPTH],
                sums.at[pl.ds((w * nrow + c) * CHUNK, CHUNK)], s2)
            nc = c + _DEPTH
            if nc < nrow:
                wrs[c].wait()
                gets[nc] = pltpu.async_copy(acc.at[rep_v.at[nc]],
                                            buf_v.at[nc % _DEPTH], s1)
        for c in range(max(0, nrow - _DEPTH), nrow):
            wrs[c].wait()


def _k2(ids2d, slot_tbl, values, zblock, ones16):
    nrow = (N // CHUNK) // NS
    return pl.kernel(
        _k2_body,
        out_type=jax.ShapeDtypeStruct((N, PAD), jnp.float32),
        mesh=plsc.VectorSubcoreMesh(**_MESH),
        compiler_params=_SC_PARAMS,
        scratch_types=[
            pltpu.VMEM_SHARED((N, PAD), jnp.float32),
            pltpu.VMEM((nrow, CHUNK), jnp.int32),
            pltpu.VMEM((nrow, CHUNK), jnp.int32),
            pltpu.VMEM((_DEPTH, CHUNK, PAD), jnp.float32),
            pltpu.SemaphoreType.DMA,
            pltpu.SemaphoreType.DMA,
            pltpu.SemaphoreType.DMA,
        ],
    )(ids2d, slot_tbl, values, zblock, ones16)


def _k3_body(pg, sums, w, fin, dec):
    s = sums[:, :P_DIM]
    cnt = sums[:, P_DIM:P_DIM + 1]
    f = (1.0 - RATE) * pg[...] + RATE * (s / cnt)
    fin[...] = f
    dec[...] = lax.dot_general(f, w[...], (((1,), (1,)), ((), ())),
                               preferred_element_type=jnp.float32)


def _k3(p_gath, sums_g, decoder_w):
    blk = 2048
    grid = N // blk
    return pl.pallas_call(
        _k3_body,
        grid=(grid,),
        in_specs=[
            pl.BlockSpec((blk, P_DIM), lambda i: (i, 0)),
            pl.BlockSpec((blk, PAD), lambda i: (i, 0)),
            pl.BlockSpec((MODEL_DIM, P_DIM), lambda i: (0, 0)),
        ],
        out_specs=(
            pl.BlockSpec((blk, P_DIM), lambda i: (i, 0)),
            pl.BlockSpec((blk, MODEL_DIM), lambda i: (i, 0)),
        ),
        out_shape=(
            jax.ShapeDtypeStruct((N, P_DIM), jnp.float32),
            jax.ShapeDtypeStruct((N, MODEL_DIM), jnp.float32),
        ),
    )(p_gath, sums_g, decoder_w)


def _k4_body(ids2d, fin, copy_in, out_ref, idx_v, row_v, gsem, ssem):
    del copy_in
    wid = lax.axis_index("c") * NS + lax.axis_index("s")
    nrow = (N // CHUNK) // (NC * NS)
    pltpu.sync_copy(ids2d.at[pl.ds(wid * nrow, nrow)], idx_v)
    gets = [pltpu.async_copy(
        fin.at[pl.ds((wid * nrow + c) * CHUNK, CHUNK)], row_v.at[c], gsem)
        for c in range(nrow)]
    puts = []
    for c in range(nrow):
        gets[c].wait()
        puts.append(pltpu.async_copy(row_v.at[c], out_ref.at[idx_v.at[c]],
                                     ssem))
    for d in puts:
        d.wait()


def _k4(ids2d, final, out0):
    nrow = (N // CHUNK) // (NC * NS)
    out, = _mpmd._mpmd_map(
        [(plsc.VectorSubcoreMesh(**_MESH), _k4_body)],
        out_types=[jax.ShapeDtypeStruct((K_MAX, P_DIM), jnp.float32)],
        input_output_aliases={2: 0},
        compiler_params=_SC_PARAMS,
        scratch_types=[
            pltpu.VMEM((nrow, CHUNK), jnp.int32),
            pltpu.VMEM((nrow, CHUNK, P_DIM), jnp.float32),
            pltpu.SemaphoreType.DMA,
            pltpu.SemaphoreType.DMA,
        ],
    )(ids2d, final, out0)
    return out


def kernel(bucket_ids, values, prototypes, decoder_w):
    ids = bucket_ids.astype(jnp.int32)
    ids2d = ids.reshape(N // CHUNK, CHUNK)
    iota2d = jnp.arange(N, dtype=jnp.int32).reshape(N // CHUNK, CHUNK)
    zblock = jnp.zeros((CHUNK, PAD), jnp.float32)
    ones16 = jnp.ones((CHUNK, PAD - P_DIM), jnp.float32)

    out0 = _k0(prototypes)
    p_gath, slot_tbl = _k1(prototypes, ids2d, iota2d)
    sums_g = _k2(ids2d, slot_tbl, values, zblock, ones16)
    final, decoded = _k3(p_gath, sums_g, decoder_w)
    new_protos = _k4(ids2d, final, out0)
    return new_protos, decoded


# speedup vs baseline: 1.3307x; 1.3307x over previous
import jax
import jax.numpy as jnp
from jax import lax
from jax.experimental import pallas as pl
from jax.experimental.pallas import tpu as pltpu
from jax.experimental.pallas import tpu_sc as plsc
from jax._src.pallas import mpmd as _mpmd

K_MAX = 100000
P_DIM = 64
MODEL_DIM = 64
N = 16384
RATE = 0.1

NC = 2
NS = 16
CHUNK = 128
PAD = 80
NCH = (N // CHUNK) // NS

_MESH = dict(core_axis_name="c", subcore_axis_name="s", num_cores=NC,
             num_subcores=NS)
_SC_PARAMS = pltpu.CompilerParams(use_tc_tiling_on_sc=False)


def _k1_body(protos, ids2d, iota2d, pgath, slot, idx_v, iot_v, row_v,
             gsem, ssem, wsem):
    wid = lax.axis_index("c") * NS + lax.axis_index("s")
    nrow = (N // CHUNK) // (NC * NS)
    pltpu.sync_copy(ids2d.at[pl.ds(wid * nrow, nrow)], idx_v)
    pltpu.sync_copy(iota2d.at[pl.ds(wid * nrow, nrow)], iot_v)
    gets = [pltpu.async_copy(protos.at[idx_v.at[c]], row_v.at[c], gsem)
            for c in range(nrow)]
    puts = [pltpu.async_copy(iot_v.at[c], slot.at[idx_v.at[c]], ssem)
            for c in range(nrow)]
    wrs = []
    for c in range(nrow):
        gets[c].wait()
        base = (wid * nrow + c) * CHUNK
        wrs.append(pltpu.async_copy(row_v.at[c], pgath.at[pl.ds(base, CHUNK)],
                                    wsem))
    for d in puts + wrs:
        d.wait()


def _k1(prototypes, ids2d, iota2d):
    nrow = (N // CHUNK) // (NC * NS)
    return pl.kernel(
        _k1_body,
        out_type=(
            jax.ShapeDtypeStruct((N, P_DIM), jnp.float32),
            jax.ShapeDtypeStruct((K_MAX,), jnp.int32),
        ),
        mesh=plsc.VectorSubcoreMesh(**_MESH),
        compiler_params=_SC_PARAMS,
        scratch_types=[
            pltpu.VMEM((nrow, CHUNK), jnp.int32),
            pltpu.VMEM((nrow, CHUNK), jnp.int32),
            pltpu.VMEM((nrow, CHUNK, P_DIM), jnp.float32),
            pltpu.SemaphoreType.DMA,
            pltpu.SemaphoreType.DMA,
            pltpu.SemaphoreType.DMA,
        ],
    )(prototypes, ids2d, iota2d)


_DEPTH = 3


def _k2_body(ids2d, slot, values, zb, ones16, sums, acc, idx_v, rep_v, buf_v,
             s1, s2, s3):
    cid = lax.axis_index("c")
    w = lax.axis_index("s")
    nrow = (N // CHUNK) // NS

    @pl.when(cid == 0)
    def _zero():
        pltpu.sync_copy(zb, buf_v.at[0])
        zs = [pltpu.async_copy(
            buf_v.at[0], acc.at[pl.ds((w * nrow + j) * CHUNK, CHUNK)], s1)
            for j in range(nrow)]
        for d in zs:
            d.wait()

    plsc.subcore_barrier()

    @pl.when(cid == 0)
    def _accum():
        for d in range(_DEPTH):
            pltpu.sync_copy(ones16, buf_v.at[d, :, pl.ds(P_DIM, PAD - P_DIM)])
        pltpu.sync_copy(ids2d.at[pl.ds(w * nrow, nrow)], idx_v)
        reps = [pltpu.async_copy(slot.at[idx_v.at[c]], rep_v.at[c], s1)
                for c in range(nrow)]
        vals = {c: pltpu.async_copy(
            values.at[pl.ds((w * nrow + c) * CHUNK, CHUNK)],
            buf_v.at[c % _DEPTH, :, pl.ds(0, P_DIM)], s2)
            for c in range(_DEPTH)}
        adds = {}
        for c in range(nrow):
            reps[c].wait()
            vals[c].wait()
            adds[c] = pltpu.async_copy(buf_v.at[c % _DEPTH],
                                       acc.at[rep_v.at[c]], s3, add=True)
            nc = c + _DEPTH
            if nc < nrow:
                adds[c].wait()
                vals[nc] = pltpu.async_copy(
                    values.at[pl.ds((w * nrow + nc) * CHUNK, CHUNK)],
                    buf_v.at[nc % _DEPTH, :, pl.ds(0, P_DIM)], s2)
        for c in range(max(0, nrow - _DEPTH), nrow):
            adds[c].wait()

    plsc.subcore_barrier()

    @pl.when(cid == 0)
    def _readback():
        gets = {c: pltpu.async_copy(acc.at[rep_v.at[c]],
                                    buf_v.at[c % _DEPTH], s1)
                for c in range(_DEPTH)}
        wrs = {}
        for c in range(nrow):
            gets[c].wait()
            wrs[c] = pltpu.async_copy(
                buf_v.at[c % _DEPTH],
                sums.at[pl.ds((w * nrow + c) * CHUNK, CHUNK)], s2)
            nc = c + _DEPTH
            if nc < nrow:
                wrs[c].wait()
                gets[nc] = pltpu.async_copy(acc.at[rep_v.at[nc]],
                                            buf_v.at[nc % _DEPTH], s1)
        for c in range(max(0, nrow - _DEPTH), nrow):
            wrs[c].wait()


def _k2(ids2d, slot_tbl, values, zblock, ones16):
    nrow = (N // CHUNK) // NS
    return pl.kernel(
        _k2_body,
        out_type=jax.ShapeDtypeStruct((N, PAD), jnp.float32),
        mesh=plsc.VectorSubcoreMesh(**_MESH),
        compiler_params=_SC_PARAMS,
        scratch_types=[
            pltpu.VMEM_SHARED((N, PAD), jnp.float32),
            pltpu.VMEM((nrow, CHUNK), jnp.int32),
            pltpu.VMEM((nrow, CHUNK), jnp.int32),
            pltpu.VMEM((_DEPTH, CHUNK, PAD), jnp.float32),
            pltpu.SemaphoreType.DMA,
            pltpu.SemaphoreType.DMA,
            pltpu.SemaphoreType.DMA,
        ],
    )(ids2d, slot_tbl, values, zblock, ones16)


def _k3_body(pg, sums, w, fin, dec):
    s = sums[:, :P_DIM]
    cnt = sums[:, P_DIM:P_DIM + 1]
    f = (1.0 - RATE) * pg[...] + RATE * (s / cnt)
    fin[...] = f
    dec[...] = lax.dot_general(f, w[...], (((1,), (1,)), ((), ())),
                               preferred_element_type=jnp.float32)


def _k3(p_gath, sums_g, decoder_w):
    blk = 2048
    grid = N // blk
    return pl.pallas_call(
        _k3_body,
        grid=(grid,),
        in_specs=[
            pl.BlockSpec((blk, P_DIM), lambda i: (i, 0)),
            pl.BlockSpec((blk, PAD), lambda i: (i, 0)),
            pl.BlockSpec((MODEL_DIM, P_DIM), lambda i: (0, 0)),
        ],
        out_specs=(
            pl.BlockSpec((blk, P_DIM), lambda i: (i, 0)),
            pl.BlockSpec((blk, MODEL_DIM), lambda i: (i, 0)),
        ),
        out_shape=(
            jax.ShapeDtypeStruct((N, P_DIM), jnp.float32),
            jax.ShapeDtypeStruct((N, MODEL_DIM), jnp.float32),
        ),
    )(p_gath, sums_g, decoder_w)


def _k4_body(ids2d, fin, copy_in, out_ref, idx_v, row_v, gsem, ssem):
    del copy_in
    wid = lax.axis_index("c") * NS + lax.axis_index("s")
    nrow = (N // CHUNK) // (NC * NS)
    pltpu.sync_copy(ids2d.at[pl.ds(wid * nrow, nrow)], idx_v)
    gets = [pltpu.async_copy(
        fin.at[pl.ds((wid * nrow + c) * CHUNK, CHUNK)], row_v.at[c], gsem)
        for c in range(nrow)]
    puts = []
    for c in range(nrow):
        gets[c].wait()
        puts.append(pltpu.async_copy(row_v.at[c], out_ref.at[idx_v.at[c]],
                                     ssem))
    for d in puts:
        d.wait()


def _k4(ids2d, final, out0):
    nrow = (N // CHUNK) // (NC * NS)
    out, = _mpmd._mpmd_map(
        [(plsc.VectorSubcoreMesh(**_MESH), _k4_body)],
        out_types=[jax.ShapeDtypeStruct((K_MAX, P_DIM), jnp.float32)],
        input_output_aliases={2: 0},
        compiler_params=_SC_PARAMS,
        scratch_types=[
            pltpu.VMEM((nrow, CHUNK), jnp.int32),
            pltpu.VMEM((nrow, CHUNK, P_DIM), jnp.float32),
            pltpu.SemaphoreType.DMA,
            pltpu.SemaphoreType.DMA,
        ],
    )(ids2d, final, out0)
    return out


def kernel(bucket_ids, values, prototypes, decoder_w):
    ids = bucket_ids.astype(jnp.int32)
    ids2d = ids.reshape(N // CHUNK, CHUNK)
    iota2d = jnp.arange(N, dtype=jnp.int32).reshape(N // CHUNK, CHUNK)
    zblock = jnp.zeros((CHUNK, PAD), jnp.float32)
    ones16 = jnp.ones((CHUNK, PAD - P_DIM), jnp.float32)

    p_gath, slot_tbl = _k1(prototypes, ids2d, iota2d)
    sums_g = _k2(ids2d, slot_tbl, values, zblock, ones16)
    final, decoded = _k3(p_gath, sums_g, decoder_w)
    new_protos = _k4(ids2d, final, prototypes)
    return new_protos, decoded


# speedup vs baseline: 1.4677x; 1.1029x over previous
import jax
import jax.numpy as jnp
from jax import lax
from jax.experimental import pallas as pl
from jax.experimental.pallas import tpu as pltpu
from jax.experimental.pallas import tpu_sc as plsc
from jax._src.pallas import mpmd as _mpmd

K_MAX = 100000
P_DIM = 64
MODEL_DIM = 64
N = 16384
RATE = 0.1

NC = 2
NS = 16
CHUNK = 128
PAD = 80
NCH = (N // CHUNK) // NS

_MESH = dict(core_axis_name="c", subcore_axis_name="s", num_cores=NC,
             num_subcores=NS)
_SC_PARAMS = pltpu.CompilerParams(use_tc_tiling_on_sc=False)


_DEPTH = 2


def _k12_body(protos, ids2d, iota2d, values, zb, ones16, pgath, sums,
              slot, acc, idxg_v, idxs_v, iot_v, rep_v, row_v, buf_v,
              s1, s2, s3):
    cid = lax.axis_index("c")
    sid = lax.axis_index("s")
    wid = cid * NS + sid
    ng = (N // CHUNK) // (NC * NS)
    nrow = (N // CHUNK) // NS

    @pl.when(cid == 0)
    def _seed():
        pltpu.sync_copy(zb, buf_v.at[0])
        zs = [pltpu.async_copy(
            buf_v.at[0], acc.at[pl.ds((sid * nrow + j) * CHUNK, CHUNK)], s3)
            for j in range(nrow)]
        pltpu.sync_copy(ids2d.at[pl.ds(sid * nrow, nrow)], idxs_v)
        pltpu.sync_copy(iota2d.at[pl.ds(sid * nrow, nrow)], iot_v)
        puts = [pltpu.async_copy(iot_v.at[c], slot.at[idxs_v.at[c]], s2)
                for c in range(nrow)]
        for d in zs + puts:
            d.wait()

    pltpu.sync_copy(ids2d.at[pl.ds(wid * ng, ng)], idxg_v)
    gets = {c: pltpu.async_copy(protos.at[idxg_v.at[c]], row_v.at[c % 2], s1)
            for c in range(min(2, ng))}
    wrs = {}
    for c in range(ng):
        gets[c].wait()
        base = (wid * ng + c) * CHUNK
        wrs[c] = pltpu.async_copy(row_v.at[c % 2],
                                  pgath.at[pl.ds(base, CHUNK)], s2)
        nc = c + 2
        if nc < ng:
            wrs[c].wait()
            gets[nc] = pltpu.async_copy(protos.at[idxg_v.at[nc]],
                                        row_v.at[nc % 2], s1)
    for c in range(max(0, ng - 2), ng):
        wrs[c].wait()

    plsc.subcore_barrier()

    @pl.when(cid == 0)
    def _accum():
        for d in range(_DEPTH):
            pltpu.sync_copy(ones16, buf_v.at[d, :, pl.ds(P_DIM, PAD - P_DIM)])
        reps = [pltpu.async_copy(slot.at[idxs_v.at[c]], rep_v.at[c], s1)
                for c in range(nrow)]
        vals = {c: pltpu.async_copy(
            values.at[pl.ds((sid * nrow + c) * CHUNK, CHUNK)],
            buf_v.at[c % _DEPTH, :, pl.ds(0, P_DIM)], s2)
            for c in range(_DEPTH)}
        adds = {}
        for c in range(nrow):
            reps[c].wait()
            vals[c].wait()
            adds[c] = pltpu.async_copy(buf_v.at[c % _DEPTH],
                                       acc.at[rep_v.at[c]], s3, add=True)
            nc = c + _DEPTH
            if nc < nrow:
                adds[c].wait()
                vals[nc] = pltpu.async_copy(
                    values.at[pl.ds((sid * nrow + nc) * CHUNK, CHUNK)],
                    buf_v.at[nc % _DEPTH, :, pl.ds(0, P_DIM)], s2)
        for c in range(max(0, nrow - _DEPTH), nrow):
            adds[c].wait()

    plsc.subcore_barrier()

    @pl.when(cid == 0)
    def _readback():
        gets = {c: pltpu.async_copy(acc.at[rep_v.at[c]],
                                    buf_v.at[c % _DEPTH], s1)
                for c in range(_DEPTH)}
        wrs = {}
        for c in range(nrow):
            gets[c].wait()
            wrs[c] = pltpu.async_copy(
                buf_v.at[c % _DEPTH],
                sums.at[pl.ds((sid * nrow + c) * CHUNK, CHUNK)], s2)
            nc = c + _DEPTH
            if nc < nrow:
                wrs[c].wait()
                gets[nc] = pltpu.async_copy(acc.at[rep_v.at[nc]],
                                            buf_v.at[nc % _DEPTH], s1)
        for c in range(max(0, nrow - _DEPTH), nrow):
            wrs[c].wait()


def _k12(prototypes, ids2d, iota2d, values, zblock, ones16):
    ng = (N // CHUNK) // (NC * NS)
    nrow = (N // CHUNK) // NS
    return pl.kernel(
        _k12_body,
        out_type=(
            jax.ShapeDtypeStruct((N, P_DIM), jnp.float32),
            jax.ShapeDtypeStruct((N, PAD), jnp.float32),
        ),
        mesh=plsc.VectorSubcoreMesh(**_MESH),
        compiler_params=_SC_PARAMS,
        scratch_types=[
            pltpu.VMEM_SHARED((K_MAX,), jnp.int32),
            pltpu.VMEM_SHARED((N, PAD), jnp.float32),
            pltpu.VMEM((ng, CHUNK), jnp.int32),
            pltpu.VMEM((nrow, CHUNK), jnp.int32),
            pltpu.VMEM((nrow, CHUNK), jnp.int32),
            pltpu.VMEM((nrow, CHUNK), jnp.int32),
            pltpu.VMEM((2, CHUNK, P_DIM), jnp.float32),
            pltpu.VMEM((_DEPTH, CHUNK, PAD), jnp.float32),
            pltpu.SemaphoreType.DMA,
            pltpu.SemaphoreType.DMA,
            pltpu.SemaphoreType.DMA,
        ],
    )(prototypes, ids2d, iota2d, values, zblock, ones16)


def _k3_body(pg, sums, w, fin, dec):
    s = sums[:, :P_DIM]
    cnt = sums[:, P_DIM:P_DIM + 1]
    f = (1.0 - RATE) * pg[...] + RATE * (s / cnt)
    fin[...] = f
    dec[...] = lax.dot_general(f, w[...], (((1,), (1,)), ((), ())),
                               preferred_element_type=jnp.float32)


def _k3(p_gath, sums_g, decoder_w):
    blk = 2048
    grid = N // blk
    return pl.pallas_call(
        _k3_body,
        grid=(grid,),
        in_specs=[
            pl.BlockSpec((blk, P_DIM), lambda i: (i, 0)),
            pl.BlockSpec((blk, PAD), lambda i: (i, 0)),
            pl.BlockSpec((MODEL_DIM, P_DIM), lambda i: (0, 0)),
        ],
        out_specs=(
            pl.BlockSpec((blk, P_DIM), lambda i: (i, 0)),
            pl.BlockSpec((blk, MODEL_DIM), lambda i: (i, 0)),
        ),
        out_shape=(
            jax.ShapeDtypeStruct((N, P_DIM), jnp.float32),
            jax.ShapeDtypeStruct((N, MODEL_DIM), jnp.float32),
        ),
    )(p_gath, sums_g, decoder_w)


def _k4_body(ids2d, fin, copy_in, out_ref, idx_v, row_v, gsem, ssem):
    del copy_in
    wid = lax.axis_index("c") * NS + lax.axis_index("s")
    nrow = (N // CHUNK) // (NC * NS)
    pltpu.sync_copy(ids2d.at[pl.ds(wid * nrow, nrow)], idx_v)
    gets = [pltpu.async_copy(
        fin.at[pl.ds((wid * nrow + c) * CHUNK, CHUNK)], row_v.at[c], gsem)
        for c in range(nrow)]
    puts = []
    for c in range(nrow):
        gets[c].wait()
        puts.append(pltpu.async_copy(row_v.at[c], out_ref.at[idx_v.at[c]],
                                     ssem))
    for d in puts:
        d.wait()


def _k4(ids2d, final, out0):
    nrow = (N // CHUNK) // (NC * NS)
    out, = _mpmd._mpmd_map(
        [(plsc.VectorSubcoreMesh(**_MESH), _k4_body)],
        out_types=[jax.ShapeDtypeStruct((K_MAX, P_DIM), jnp.float32)],
        input_output_aliases={2: 0},
        compiler_params=_SC_PARAMS,
        scratch_types=[
            pltpu.VMEM((nrow, CHUNK), jnp.int32),
            pltpu.VMEM((nrow, CHUNK, P_DIM), jnp.float32),
            pltpu.SemaphoreType.DMA,
            pltpu.SemaphoreType.DMA,
        ],
    )(ids2d, final, out0)
    return out


def kernel(bucket_ids, values, prototypes, decoder_w):
    ids = bucket_ids.astype(jnp.int32)
    ids2d = ids.reshape(N // CHUNK, CHUNK)
    iota2d = jnp.arange(N, dtype=jnp.int32).reshape(N // CHUNK, CHUNK)
    zblock = jnp.zeros((CHUNK, PAD), jnp.float32)
    ones16 = jnp.ones((CHUNK, PAD - P_DIM), jnp.float32)

    p_gath, sums_g = _k12(prototypes, ids2d, iota2d, values, zblock, ones16)
    final, decoded = _k3(p_gath, sums_g, decoder_w)
    new_protos = _k4(ids2d, final, prototypes)
    return new_protos, decoded
